# baseline (speedup 1.0000x reference)
import functools

import jax
import jax.numpy as jnp
from jax import lax
from jax.experimental import pallas as pl
from jax.experimental.pallas import tpu as pltpu
from jax.experimental.pallas import tpu_sc as plsc

N_NODES = 10000
N_EDGES = 320000
D_IN = 128
D_HID = 64

NPAD = 10240
EPAD = 327680
DUMMY = NPAD - 1
NCH = 80
NB = 2
EPT = EPAD // 16
EROW = EPT // 128
NROW = NPAD // 128

_f32 = jnp.float32
_i32 = jnp.int32


def _vsc_mesh():
    return plsc.VectorSubcoreMesh(core_axis_name="c", subcore_axis_name="s")


def _zero_2d(ref, nrows):
    zero = jnp.zeros((16,), _f32)
    ngrp = ref.shape[1] // 16

    def body(i, _):
        for j in range(ngrp):
            ref[i, pl.ds(j * 16, 16)] = zero
        return 0

    lax.fori_loop(0, nrows, body, 0)


def _fill_iota_row(idx_ref, n):
    for j in range(n // 16):
        idx_ref[0, pl.ds(j * 16, 16)] = lax.iota(_i32, 16) + j * 16


def _newton_rsqrt(d):
    i = plsc.bitcast(d, _i32)
    i = jnp.int32(0x5F3759DF) - (i >> 1)
    r = plsc.bitcast(i, _f32)
    for _ in range(3):
        r = r * (1.5 - 0.5 * d * r * r)
    return r


def _k1_body(e_hbm, dinv_hbm, dv_v, hist_v, idx_v, degl_v, dinv_v, deg_sp):
    c = lax.axis_index("c")
    s = lax.axis_index("s")

    @pl.when(c == 0)
    def _():
        _zero_2d(hist_v, NROW)

        @pl.when(s == 0)
        def _():
            pltpu.sync_copy(hist_v, deg_sp)

        _fill_iota_row(idx_v, NROW)
        pltpu.sync_copy(e_hbm.at[1, pl.ds(s * EROW, EROW)], dv_v)
        plsc.subcore_barrier()

        ones = jnp.ones((16,), _f32)

        def body(k, _):
            dv = dv_v[k >> 3, pl.ds((k & 7) * 16, 16)]
            plsc.addupdate_scatter(hist_v, [dv >> 7, dv & 127], ones)
            return 0

        lax.fori_loop(0, EPT // 16, body, 0)

        pltpu.sync_copy(hist_v, deg_sp.at[idx_v.at[0]], add=True)
        plsc.subcore_barrier()

        pltpu.sync_copy(deg_sp.at[pl.ds(s * 5, 5)], degl_v)

        def fin(k, _):
            r = k >> 3
            cg = k & 7
            d = degl_v[r, pl.ds(cg * 16, 16)] + 1.0
            dinv_v[pl.ds(k * 16, 16)] = _newton_rsqrt(d)
            return 0

        lax.fori_loop(0, 40, fin, 0)
        pltpu.sync_copy(dinv_v, dinv_hbm.at[pl.ds(s * 640, 640)])


@jax.jit
def _k1(e3):
    return pl.kernel(
        _k1_body,
        out_type=jax.ShapeDtypeStruct((NPAD,), _f32),
        mesh=_vsc_mesh(),
        compiler_params=pltpu.CompilerParams(needs_layout_passes=False,
                                             use_tc_tiling_on_sc=False),
        scratch_types=[
            pltpu.VMEM((EROW, 128), _i32),
            pltpu.VMEM((NROW, 128), _f32),
            pltpu.VMEM((1, NROW), _i32),
            pltpu.VMEM((5, 128), _f32),
            pltpu.VMEM((640,), _f32),
            pltpu.VMEM_SHARED((NROW, 128), _f32),
        ],
    )(e3)


def _k2_body(x_ref, w_ref, dinv_ref, y_ref):
    xw = jnp.dot(x_ref[...], w_ref[...], preferred_element_type=_f32)
    y_ref[:N_NODES, :] = xw * dinv_ref[...]
    y_ref[N_NODES:, :] = jnp.zeros((NPAD - N_NODES, D_HID), _f32)


@jax.jit
def _k2(x, W1, dinv_col):
    return pl.pallas_call(
        _k2_body,
        out_shape=jax.ShapeDtypeStruct((NPAD, D_HID), _f32),
    )(x, W1, dinv_col)


def _k3_body(y_hbm, e_hbm, out_hbm, src_v, dst_v, *rest):
    rows = list(rest[:NB])
    gsems = list(rest[NB:2 * NB])
    ssems = list(rest[2 * NB:3 * NB])
    acc_sp = rest[3 * NB]
    y_sp = rest[3 * NB + 1]
    c = lax.axis_index("c")
    s = lax.axis_index("s")
    w = c * 16 + s

    pltpu.sync_copy(y_hbm.at[pl.ds(s * 640, 640)],
                    y_sp.at[pl.ds(s * 640, 640)])

    _zero_2d(rows[0], 128)
    for t in range(5):
        pltpu.sync_copy(rows[0], acc_sp.at[pl.ds(s * 640 + t * 128, 128)])

    pltpu.sync_copy(e_hbm.at[0, pl.ds(w * NCH, NCH)], src_v)
    pltpu.sync_copy(e_hbm.at[1, pl.ds(w * NCH, NCH)], dst_v)
    plsc.subcore_barrier()

    for b in range(NB):
        pltpu.async_copy(y_sp.at[src_v.at[b]], rows[b], gsems[b])

    def outer(g, _):
        base = g * NB
        for b in range(NB):
            ch = base + b
            pltpu.make_async_copy(y_sp.at[src_v.at[ch]], rows[b],
                                  gsems[b]).wait()
            pltpu.async_copy(rows[b], acc_sp.at[dst_v.at[ch]], ssems[b],
                             add=True)
        for b in range(NB):
            ch = base + b
            nxt = ch + NB

            @pl.when(nxt < NCH)
            def _():
                pltpu.make_async_copy(rows[b], acc_sp.at[dst_v.at[ch]],
                                      ssems[b]).wait()
                pltpu.async_copy(y_sp.at[src_v.at[nxt]], rows[b], gsems[b])

        return 0

    lax.fori_loop(0, NCH // NB, outer, 0)

    for b in range(NB):
        ch = NCH - NB + b
        pltpu.make_async_copy(rows[b], acc_sp.at[dst_v.at[ch]],
                              ssems[b]).wait()

    plsc.subcore_barrier()
    pltpu.sync_copy(acc_sp.at[pl.ds(s * 640, 640)],
                    out_hbm.at[c, pl.ds(s * 640, 640)])


@jax.jit
def _k3(y, e3):
    return pl.kernel(
        _k3_body,
        out_type=jax.ShapeDtypeStruct((2, NPAD, D_HID), _f32),
        mesh=_vsc_mesh(),
        compiler_params=pltpu.CompilerParams(needs_layout_passes=False,
                                             use_tc_tiling_on_sc=False),
        scratch_types=(
            [pltpu.VMEM((NCH, 128), _i32)] * 2
            + [pltpu.VMEM((128, D_HID), _f32)] * NB
            + [pltpu.SemaphoreType.DMA] * (2 * NB)
            + [pltpu.VMEM_SHARED((NPAD, D_HID), _f32)] * 2
        ),
    )(y, e3)


def _k4_body(p_ref, y_ref, dinv_ref, w2_ref, b1_ref, z_ref):
    agg = p_ref[0] + p_ref[1]
    h = jnp.maximum((agg + y_ref[...]) * dinv_ref[...] + b1_ref[...], 0.0)
    z_ref[...] = jnp.dot(h, w2_ref[...], preferred_element_type=_f32) \
        * dinv_ref[...]


@jax.jit
def _k4(parts, y, dinv_col, W2, b1_row):
    return pl.pallas_call(
        _k4_body,
        out_shape=jax.ShapeDtypeStruct((NPAD, 1), _f32),
    )(parts, y, dinv_col, W2, b1_row)


def _k5_body(e_hbm, z_hbm, dinv_hbm, b2_hbm, out_hbm,
             sv_v, dv_v, z_v, acc_v, idx_v, b2_v, dinv_v, aggl_v, out_v,
             agg_sp):
    c = lax.axis_index("c")
    s = lax.axis_index("s")

    @pl.when(c == 0)
    def _():
        _zero_2d(acc_v, NROW)

        @pl.when(s == 0)
        def _():
            pltpu.sync_copy(acc_v, agg_sp)

        _fill_iota_row(idx_v, NROW)
        pltpu.sync_copy(e_hbm.at[0, pl.ds(s * EROW, EROW)], sv_v)
        pltpu.sync_copy(e_hbm.at[1, pl.ds(s * EROW, EROW)], dv_v)
        pltpu.sync_copy(z_hbm, z_v)
        pltpu.sync_copy(b2_hbm, b2_v)
        plsc.subcore_barrier()

        def body(k, _):
            r = k >> 3
            sl = pl.ds((k & 7) * 16, 16)
            sv = sv_v[r, sl]
            dv = dv_v[r, sl]
            vals = plsc.load_gather(z_v, [sv])
            plsc.addupdate_scatter(acc_v, [dv >> 7, dv & 127], vals)
            return 0

        lax.fori_loop(0, EPT // 16, body, 0)

        pltpu.sync_copy(acc_v, agg_sp.at[idx_v.at[0]], add=True)
        plsc.subcore_barrier()

        pltpu.sync_copy(agg_sp.at[pl.ds(s * 5, 5)], aggl_v)
        pltpu.sync_copy(dinv_hbm.at[pl.ds(s * 640, 640)], dinv_v)
        b2 = b2_v[...]

        def fin(k, _):
            r = k >> 3
            cg = k & 7
            sl = pl.ds(k * 16, 16)
            a = aggl_v[r, pl.ds(cg * 16, 16)]
            zz = z_v[pl.ds(s * 640 + k * 16, 16)]
            out_v[sl] = dinv_v[sl] * (a + zz) + b2
            return 0

        lax.fori_loop(0, 40, fin, 0)
        pltpu.sync_copy(out_v, out_hbm.at[pl.ds(s * 640, 640)])


@jax.jit
def _k5(e3, z_flat, dinv, b2v):
    return pl.kernel(
        _k5_body,
        out_type=jax.ShapeDtypeStruct((NPAD,), _f32),
        mesh=_vsc_mesh(),
        compiler_params=pltpu.CompilerParams(needs_layout_passes=False,
                                             use_tc_tiling_on_sc=False),
        scratch_types=[
            pltpu.VMEM((EROW, 128), _i32),
            pltpu.VMEM((EROW, 128), _i32),
            pltpu.VMEM((NPAD,), _f32),
            pltpu.VMEM((NROW, 128), _f32),
            pltpu.VMEM((1, NROW), _i32),
            pltpu.VMEM((16,), _f32),
            pltpu.VMEM((640,), _f32),
            pltpu.VMEM((5, 128), _f32),
            pltpu.VMEM((640,), _f32),
            pltpu.VMEM_SHARED((NROW, 128), _f32),
        ],
    )(e3, z_flat, dinv, b2v)


def kernel(x, edges, W1, b1, W2, b2):
    fill = N_NODES + (lax.iota(_i32, EPAD - N_EDGES) % (NPAD - N_NODES))
    ep = jnp.concatenate([edges.T, jnp.stack([fill, fill])], axis=1)
    e3 = ep.reshape(2, EPAD // 128, 128)

    dinv = _k1(e3)
    y = _k2(x, W1, dinv[:N_NODES, None])
    parts = _k3(y, e3)
    z = _k4(parts, y, dinv[:, None], W2, b1.reshape(1, D_HID))
    b2v = jnp.broadcast_to(b2, (16,)).astype(_f32)
    out2 = _k5(e3, z.reshape(NPAD), dinv, b2v)
    return out2[:N_NODES, None]


# speedup vs baseline: 1.2075x; 1.2075x over previous
import functools

import jax
import jax.numpy as jnp
from jax import lax
from jax.experimental import pallas as pl
from jax.experimental.pallas import tpu as pltpu
from jax.experimental.pallas import tpu_sc as plsc

N_NODES = 10000
N_EDGES = 320000
D_IN = 128
D_HID = 64

NPAD = 10240
EPAD = 327680
DUMMY = NPAD - 1
NCH = 80
NB = 5
EPT = EPAD // 16
EROW = EPT // 128
E32 = EPAD // 32 // 128
NROW = NPAD // 128

_f32 = jnp.float32
_i32 = jnp.int32


def _vsc_mesh():
    return plsc.VectorSubcoreMesh(core_axis_name="c", subcore_axis_name="s")


def _zero_2d(ref, nrows):
    zero = jnp.zeros((16,), _f32)
    ngrp = ref.shape[1] // 16

    def body(i, _):
        for j in range(ngrp):
            ref[i, pl.ds(j * 16, 16)] = zero
        return 0

    lax.fori_loop(0, nrows, body, 0)


def _fill_iota_row(idx_ref, n):
    for j in range(n // 16):
        idx_ref[0, pl.ds(j * 16, 16)] = lax.iota(_i32, 16) + j * 16


def _newton_rsqrt(d):
    i = plsc.bitcast(d, _i32)
    i = jnp.int32(0x5F3759DF) - (i >> 1)
    r = plsc.bitcast(i, _f32)
    for _ in range(3):
        r = r * (1.5 - 0.5 * d * r * r)
    return r


def _k1_body(e_hbm, dinv_hbm, dv_v, hist_v, idx_v, degl_v, dinv_v, deg_sp):
    c = lax.axis_index("c")
    s = lax.axis_index("s")

    @pl.when(c == 0)
    def _():
        _zero_2d(hist_v, NROW)

        @pl.when(s == 0)
        def _():
            pltpu.sync_copy(hist_v, deg_sp)

        _fill_iota_row(idx_v, NROW)
        pltpu.sync_copy(e_hbm.at[1, pl.ds(s * EROW, EROW)], dv_v)
        plsc.subcore_barrier()

        ones = jnp.ones((16,), _f32)

        def body(k, _):
            dv = dv_v[k >> 3, pl.ds((k & 7) * 16, 16)]
            plsc.addupdate_scatter(hist_v, [dv >> 7, dv & 127], ones)
            return 0

        lax.fori_loop(0, EPT // 16, body, 0)

        pltpu.sync_copy(hist_v, deg_sp.at[idx_v.at[0]], add=True)
        plsc.subcore_barrier()

        pltpu.sync_copy(deg_sp.at[pl.ds(s * 5, 5)], degl_v)

        def fin(k, _):
            r = k >> 3
            cg = k & 7
            d = degl_v[r, pl.ds(cg * 16, 16)] + 1.0
            dinv_v[pl.ds(k * 16, 16)] = _newton_rsqrt(d)
            return 0

        lax.fori_loop(0, 40, fin, 0)
        pltpu.sync_copy(dinv_v, dinv_hbm.at[pl.ds(s * 640, 640)])


@jax.jit
def _k1(e3):
    return pl.kernel(
        _k1_body,
        out_type=jax.ShapeDtypeStruct((NPAD,), _f32),
        mesh=_vsc_mesh(),
        compiler_params=pltpu.CompilerParams(needs_layout_passes=False,
                                             use_tc_tiling_on_sc=False),
        scratch_types=[
            pltpu.VMEM((EROW, 128), _i32),
            pltpu.VMEM((NROW, 128), _f32),
            pltpu.VMEM((1, NROW), _i32),
            pltpu.VMEM((5, 128), _f32),
            pltpu.VMEM((640,), _f32),
            pltpu.VMEM_SHARED((NROW, 128), _f32),
        ],
    )(e3)


def _k2_body(x_ref, w_ref, dc_ref, y_ref):
    xw = jnp.dot(x_ref[...], w_ref[...], preferred_element_type=_f32)
    d = dc_ref[...]
    y_ref[:N_NODES, :] = xw * d[:N_NODES]
    y_ref[N_NODES:, :] = jnp.zeros((NPAD - N_NODES, D_HID), _f32)


@jax.jit
def _k2(x, W1, dcol):
    return pl.pallas_call(
        _k2_body,
        out_shape=jax.ShapeDtypeStruct((NPAD, D_HID), _f32),
    )(x, W1, dcol)


def _k3_body(y_hbm, e_hbm, out_hbm, src_v, dst_v, *rest):
    rows = list(rest[:NB])
    gsems = list(rest[NB:2 * NB])
    ssems = list(rest[2 * NB:3 * NB])
    acc_sp = rest[3 * NB]
    c = lax.axis_index("c")
    s = lax.axis_index("s")
    w = c * 16 + s

    _zero_2d(rows[0], 128)
    for t in range(5):
        pltpu.sync_copy(rows[0], acc_sp.at[pl.ds(s * 640 + t * 128, 128)])

    pltpu.sync_copy(e_hbm.at[0, pl.ds(w * NCH, NCH)], src_v)
    pltpu.sync_copy(e_hbm.at[1, pl.ds(w * NCH, NCH)], dst_v)
    plsc.subcore_barrier()

    for b in range(NB):
        pltpu.async_copy(y_hbm.at[src_v.at[b]], rows[b], gsems[b])

    def outer(g, _):
        base = g * NB
        for b in range(NB):
            ch = base + b
            pltpu.make_async_copy(y_hbm.at[src_v.at[ch]], rows[b],
                                  gsems[b]).wait()
            pltpu.async_copy(rows[b], acc_sp.at[dst_v.at[ch]], ssems[b],
                             add=True)
        for b in range(NB):
            ch = base + b
            nxt = ch + NB

            @pl.when(nxt < NCH)
            def _():
                pltpu.make_async_copy(rows[b], acc_sp.at[dst_v.at[ch]],
                                      ssems[b]).wait()
                pltpu.async_copy(y_hbm.at[src_v.at[nxt]], rows[b], gsems[b])

        return 0

    lax.fori_loop(0, NCH // NB, outer, 0)

    for b in range(NB):
        ch = NCH - NB + b
        pltpu.make_async_copy(rows[b], acc_sp.at[dst_v.at[ch]],
                              ssems[b]).wait()

    plsc.subcore_barrier()
    pltpu.sync_copy(acc_sp.at[pl.ds(s * 640, 640)],
                    out_hbm.at[c, pl.ds(s * 640, 640)])


@jax.jit
def _k3(y, e3):
    return pl.kernel(
        _k3_body,
        out_type=jax.ShapeDtypeStruct((2, NPAD, D_HID), _f32),
        mesh=_vsc_mesh(),
        compiler_params=pltpu.CompilerParams(needs_layout_passes=False,
                                             use_tc_tiling_on_sc=False),
        scratch_types=(
            [pltpu.VMEM((NCH, 128), _i32)] * 2
            + [pltpu.VMEM((128, D_HID), _f32)] * NB
            + [pltpu.SemaphoreType.DMA] * (2 * NB)
            + [pltpu.VMEM_SHARED((NPAD, D_HID), _f32)]
        ),
    )(y, e3)


def _k4_body(p_ref, y_ref, dc_ref, w2_ref, b1_ref, z_ref):
    d = dc_ref[...]
    agg = p_ref[0] + p_ref[1]
    h = jnp.maximum((agg + y_ref[...]) * d + b1_ref[...], 0.0)
    z_ref[...] = jnp.dot(h, w2_ref[...], preferred_element_type=_f32) * d


@jax.jit
def _k4(parts, y, dcol, W2, b1_row):
    return pl.pallas_call(
        _k4_body,
        out_shape=jax.ShapeDtypeStruct((NPAD, 1), _f32),
    )(parts, y, dcol, W2, b1_row)


def _k5_body(e_hbm, z_hbm, out_hbm, sv_v, dv_v, z_v, acc_v, idx_v, agg_sp):
    c = lax.axis_index("c")
    s = lax.axis_index("s")
    w = c * 16 + s

    _zero_2d(acc_v, NROW)

    @pl.when(s == 0)
    def _():
        pltpu.sync_copy(acc_v, agg_sp)

    _fill_iota_row(idx_v, NROW)
    pltpu.sync_copy(e_hbm.at[0, pl.ds(w * E32, E32)], sv_v)
    pltpu.sync_copy(e_hbm.at[1, pl.ds(w * E32, E32)], dv_v)
    pltpu.sync_copy(z_hbm, z_v)
    plsc.subcore_barrier()

    def body(k, _):
        r = k >> 3
        sl = pl.ds((k & 7) * 16, 16)
        sv = sv_v[r, sl]
        dv = dv_v[r, sl]
        vals = plsc.load_gather(z_v, [sv >> 7, sv & 127])
        plsc.addupdate_scatter(acc_v, [dv >> 7, dv & 127], vals)
        return 0

    lax.fori_loop(0, (E32 * 128) // 16, body, 0)

    pltpu.sync_copy(acc_v, agg_sp.at[idx_v.at[0]], add=True)
    plsc.subcore_barrier()

    pltpu.sync_copy(agg_sp.at[pl.ds(s * 5, 5)],
                    out_hbm.at[c, pl.ds(s * 5, 5)])


@jax.jit
def _k5(e3, z2):
    return pl.kernel(
        _k5_body,
        out_type=jax.ShapeDtypeStruct((2, NROW, 128), _f32),
        mesh=_vsc_mesh(),
        compiler_params=pltpu.CompilerParams(needs_layout_passes=False,
                                             use_tc_tiling_on_sc=False),
        scratch_types=[
            pltpu.VMEM((E32, 128), _i32),
            pltpu.VMEM((E32, 128), _i32),
            pltpu.VMEM((NROW, 128), _f32),
            pltpu.VMEM((NROW, 128), _f32),
            pltpu.VMEM((1, NROW), _i32),
            pltpu.VMEM_SHARED((NROW, 128), _f32),
        ],
    )(e3, z2)


def kernel(x, edges, W1, b1, W2, b2):
    fill = N_NODES + (lax.iota(_i32, EPAD - N_EDGES) % (NPAD - N_NODES))
    ep = jnp.concatenate([edges.T, jnp.stack([fill, fill])], axis=1)
    e3 = ep.reshape(2, EPAD // 128, 128)

    dinv = _k1(e3)
    dcol = dinv.reshape(NPAD, 1)
    d2 = dinv.reshape(NROW, 128)
    y = _k2(x, W1, dcol)
    parts = _k3(y, e3)
    z = _k4(parts, y, dcol, W2, b1.reshape(1, D_HID))
    z2 = z.reshape(NROW, 128)
    p5 = _k5(e3, z2)
    out2 = d2 * (p5[0] + p5[1] + z2) + b2
    return out2.reshape(NPAD)[:N_NODES, None]


# speedup vs baseline: 1.2078x; 1.0003x over previous
import functools

import jax
import jax.numpy as jnp
from jax import lax
from jax.experimental import pallas as pl
from jax.experimental.pallas import tpu as pltpu
from jax.experimental.pallas import tpu_sc as plsc

N_NODES = 10000
N_EDGES = 320000
D_IN = 128
D_HID = 64

NPAD = 10240
EPAD = 327680
DUMMY = NPAD - 1
NCH = 80
NB = 5
EPT = EPAD // 16
EROW = EPT // 128
E32 = EPAD // 32 // 128
NROW = NPAD // 128

_f32 = jnp.float32
_i32 = jnp.int32


def _vsc_mesh():
    return plsc.VectorSubcoreMesh(core_axis_name="c", subcore_axis_name="s")


def _zero_2d(ref, nrows):
    zero = jnp.zeros((16,), _f32)
    ngrp = ref.shape[1] // 16

    def body(i, _):
        for j in range(ngrp):
            ref[i, pl.ds(j * 16, 16)] = zero
        return 0

    lax.fori_loop(0, nrows, body, 0)


def _fill_iota_row(idx_ref, n):
    for j in range(n // 16):
        idx_ref[0, pl.ds(j * 16, 16)] = lax.iota(_i32, 16) + j * 16


def _newton_rsqrt(d):
    i = plsc.bitcast(d, _i32)
    i = jnp.int32(0x5F3759DF) - (i >> 1)
    r = plsc.bitcast(i, _f32)
    for _ in range(3):
        r = r * (1.5 - 0.5 * d * r * r)
    return r


def _k1_body(e_hbm, dinv_hbm, dv_v, hist_v, idx_v, degl_v, dinv_v, deg_sp):
    c = lax.axis_index("c")
    s = lax.axis_index("s")

    @pl.when(c == 0)
    def _():
        _zero_2d(hist_v, NROW)

        @pl.when(s == 0)
        def _():
            pltpu.sync_copy(hist_v, deg_sp)

        _fill_iota_row(idx_v, NROW)
        pltpu.sync_copy(e_hbm.at[1, pl.ds(s * EROW, EROW)], dv_v)
        plsc.subcore_barrier()

        ones = jnp.ones((16,), _f32)

        def body(k, _):
            dv = dv_v[k >> 3, pl.ds((k & 7) * 16, 16)]
            plsc.addupdate_scatter(hist_v, [dv >> 7, dv & 127], ones)
            return 0

        lax.fori_loop(0, EPT // 16, body, 0)

        pltpu.sync_copy(hist_v, deg_sp.at[idx_v.at[0]], add=True)
        plsc.subcore_barrier()

        pltpu.sync_copy(deg_sp.at[pl.ds(s * 5, 5)], degl_v)

        def fin(k, _):
            r = k >> 3
            cg = k & 7
            d = degl_v[r, pl.ds(cg * 16, 16)] + 1.0
            dinv_v[pl.ds(k * 16, 16)] = _newton_rsqrt(d)
            return 0

        lax.fori_loop(0, 40, fin, 0)
        pltpu.sync_copy(dinv_v, dinv_hbm.at[pl.ds(s * 640, 640)])


@jax.jit
def _k1(e3):
    return pl.kernel(
        _k1_body,
        out_type=jax.ShapeDtypeStruct((NPAD,), _f32),
        mesh=_vsc_mesh(),
        compiler_params=pltpu.CompilerParams(needs_layout_passes=False,
                                             use_tc_tiling_on_sc=False),
        scratch_types=[
            pltpu.VMEM((EROW, 128), _i32),
            pltpu.VMEM((NROW, 128), _f32),
            pltpu.VMEM((1, NROW), _i32),
            pltpu.VMEM((5, 128), _f32),
            pltpu.VMEM((640,), _f32),
            pltpu.VMEM_SHARED((NROW, 128), _f32),
        ],
    )(e3)


def _k2a_body(x_ref, w_ref, o_ref):
    o_ref[...] = jnp.dot(x_ref[...], w_ref[...], preferred_element_type=_f32)


@jax.jit
def _k2a(x, W1):
    return pl.pallas_call(
        _k2a_body,
        out_shape=jax.ShapeDtypeStruct((N_NODES, D_HID), _f32),
    )(x, W1)


def _k2b_body(xw_ref, dc_ref, y_ref):
    d = dc_ref[...]
    y_ref[:N_NODES, :] = xw_ref[...] * d[:N_NODES]
    y_ref[N_NODES:, :] = jnp.zeros((NPAD - N_NODES, D_HID), _f32)


@jax.jit
def _k2b(xw, dcol):
    return pl.pallas_call(
        _k2b_body,
        out_shape=jax.ShapeDtypeStruct((NPAD, D_HID), _f32),
    )(xw, dcol)


def _k3_body(y_hbm, e_hbm, out_hbm, src_v, dst_v, *rest):
    rows = list(rest[:NB])
    gsems = list(rest[NB:2 * NB])
    ssems = list(rest[2 * NB:3 * NB])
    acc_sp = rest[3 * NB]
    c = lax.axis_index("c")
    s = lax.axis_index("s")
    w = c * 16 + s

    _zero_2d(rows[0], 128)
    for t in range(5):
        pltpu.sync_copy(rows[0], acc_sp.at[pl.ds(s * 640 + t * 128, 128)])

    pltpu.sync_copy(e_hbm.at[0, pl.ds(w * NCH, NCH)], src_v)
    pltpu.sync_copy(e_hbm.at[1, pl.ds(w * NCH, NCH)], dst_v)
    plsc.subcore_barrier()

    for b in range(NB):
        pltpu.async_copy(y_hbm.at[src_v.at[b]], rows[b], gsems[b])

    def outer(g, _):
        base = g * NB
        for b in range(NB):
            ch = base + b
            pltpu.make_async_copy(y_hbm.at[src_v.at[ch]], rows[b],
                                  gsems[b]).wait()
            pltpu.async_copy(rows[b], acc_sp.at[dst_v.at[ch]], ssems[b],
                             add=True)
        for b in range(NB):
            ch = base + b
            nxt = ch + NB

            @pl.when(nxt < NCH)
            def _():
                pltpu.make_async_copy(rows[b], acc_sp.at[dst_v.at[ch]],
                                      ssems[b]).wait()
                pltpu.async_copy(y_hbm.at[src_v.at[nxt]], rows[b], gsems[b])

        return 0

    lax.fori_loop(0, NCH // NB, outer, 0)

    for b in range(NB):
        ch = NCH - NB + b
        pltpu.make_async_copy(rows[b], acc_sp.at[dst_v.at[ch]],
                              ssems[b]).wait()

    plsc.subcore_barrier()
    pltpu.sync_copy(acc_sp.at[pl.ds(s * 640, 640)],
                    out_hbm.at[c, pl.ds(s * 640, 640)])


@jax.jit
def _k3(y, e3):
    return pl.kernel(
        _k3_body,
        out_type=jax.ShapeDtypeStruct((2, NPAD, D_HID), _f32),
        mesh=_vsc_mesh(),
        compiler_params=pltpu.CompilerParams(needs_layout_passes=False,
                                             use_tc_tiling_on_sc=False),
        scratch_types=(
            [pltpu.VMEM((NCH, 128), _i32)] * 2
            + [pltpu.VMEM((128, D_HID), _f32)] * NB
            + [pltpu.SemaphoreType.DMA] * (2 * NB)
            + [pltpu.VMEM_SHARED((NPAD, D_HID), _f32)]
        ),
    )(y, e3)


def _k4_body(p_ref, y_ref, dc_ref, w2_ref, b1_ref, z_ref):
    d = dc_ref[...]
    agg = p_ref[0] + p_ref[1]
    h = jnp.maximum((agg + y_ref[...]) * d + b1_ref[...], 0.0)
    z_ref[...] = jnp.dot(h, w2_ref[...], preferred_element_type=_f32) * d


@jax.jit
def _k4(parts, y, dcol, W2, b1_row):
    return pl.pallas_call(
        _k4_body,
        out_shape=jax.ShapeDtypeStruct((NPAD, 1), _f32),
    )(parts, y, dcol, W2, b1_row)


def _k5_body(e_hbm, z_hbm, out_hbm, sv_v, dv_v, z_v, acc_v, idx_v, agg_sp):
    c = lax.axis_index("c")
    s = lax.axis_index("s")
    w = c * 16 + s

    _zero_2d(acc_v, NROW)

    @pl.when(s == 0)
    def _():
        pltpu.sync_copy(acc_v, agg_sp)

    _fill_iota_row(idx_v, NROW)
    pltpu.sync_copy(e_hbm.at[0, pl.ds(w * E32, E32)], sv_v)
    pltpu.sync_copy(e_hbm.at[1, pl.ds(w * E32, E32)], dv_v)
    pltpu.sync_copy(z_hbm, z_v)
    plsc.subcore_barrier()

    def body(k, _):
        r = k >> 3
        sl = pl.ds((k & 7) * 16, 16)
        sv = sv_v[r, sl]
        dv = dv_v[r, sl]
        vals = plsc.load_gather(z_v, [sv >> 7, sv & 127])
        plsc.addupdate_scatter(acc_v, [dv >> 7, dv & 127], vals)
        return 0

    lax.fori_loop(0, (E32 * 128) // 16, body, 0)

    pltpu.sync_copy(acc_v, agg_sp.at[idx_v.at[0]], add=True)
    plsc.subcore_barrier()

    pltpu.sync_copy(agg_sp.at[pl.ds(s * 5, 5)],
                    out_hbm.at[c, pl.ds(s * 5, 5)])


@jax.jit
def _k5(e3, z2):
    return pl.kernel(
        _k5_body,
        out_type=jax.ShapeDtypeStruct((2, NROW, 128), _f32),
        mesh=_vsc_mesh(),
        compiler_params=pltpu.CompilerParams(needs_layout_passes=False,
                                             use_tc_tiling_on_sc=False),
        scratch_types=[
            pltpu.VMEM((E32, 128), _i32),
            pltpu.VMEM((E32, 128), _i32),
            pltpu.VMEM((NROW, 128), _f32),
            pltpu.VMEM((NROW, 128), _f32),
            pltpu.VMEM((1, NROW), _i32),
            pltpu.VMEM_SHARED((NROW, 128), _f32),
        ],
    )(e3, z2)


def kernel(x, edges, W1, b1, W2, b2):
    fill = N_NODES + (lax.iota(_i32, EPAD - N_EDGES) % (NPAD - N_NODES))
    ep = jnp.concatenate([edges.T, jnp.stack([fill, fill])], axis=1)
    e3 = ep.reshape(2, EPAD // 128, 128)

    xw = _k2a(x, W1)
    dinv = _k1(e3)
    dcol = dinv.reshape(NPAD, 1)
    d2 = dinv.reshape(NROW, 128)
    y = _k2b(xw, dcol)
    parts = _k3(y, e3)
    z = _k4(parts, y, dcol, W2, b1.reshape(1, D_HID))
    z2 = z.reshape(NROW, 128)
    p5 = _k5(e3, z2)
    out2 = d2 * (p5[0] + p5[1] + z2) + b2
    return out2.reshape(NPAD)[:N_NODES, None]
